# baseline (speedup 1.0000x reference)
import functools

import jax
import jax.numpy as jnp
from jax import lax
from jax.experimental import pallas as pl

_EMPHASIS = 1.5
_TOPK_FRAC = 0.2


def _attn_body(k_value, idx_bits, Wt_ref, b_ref, x_ref, sum_ref, emph_ref):
    int_min = jnp.int32(-(2 ** 31))
    xb = x_ref[0]
    Wt = Wt_ref[...]
    T = xb.shape[0]

    pre = lax.dot_general(
        Wt, xb, dimension_numbers=(((1,), (1,)), ((), ())),
        preferred_element_type=jnp.float32)
    e = jnp.tanh(pre + b_ref[0, 0])
    ex = jnp.exp(e)
    z = jnp.sum(ex)

    bits = lax.bitcast_convert_type(e, jnp.int32)
    skey = jnp.where(bits >= 0, bits, bits ^ jnp.int32(0x7FFFFFFF))
    skey = jnp.where(bits == int_min, jnp.int32(0), skey)

    def val_round(i, p_u):
        bit = lax.shift_left(jnp.int32(1), jnp.int32(31) - i)
        cand = p_u | bit
        scand = cand ^ int_min
        c = jnp.sum((skey >= scand).astype(jnp.int32))
        return jnp.where(c >= k_value, cand, p_u)

    p_u = lax.fori_loop(0, 32, val_round, jnp.int32(0))
    s_star = p_u ^ int_min

    gt = skey > s_star
    g = jnp.sum(gt.astype(jnp.int32))
    r = k_value - g
    eq = skey == s_star
    idx = lax.broadcasted_iota(jnp.int32, (1, T), 1)

    def idx_round(j, p):
        bit = lax.shift_left(jnp.int32(1), jnp.int32(idx_bits - 1) - j)
        t_test = p | (bit - 1)
        f = jnp.sum((eq & (idx <= t_test)).astype(jnp.int32))
        return jnp.where(f >= r, p, p | bit)

    t_star = lax.fori_loop(0, idx_bits, idx_round, jnp.int32(0))
    mask = gt | (eq & (idx <= t_star))

    w = jnp.where(mask, ex * jnp.float32(_EMPHASIS), ex) / z
    emph_ref[0] = w
    summed = lax.dot_general(
        w, xb, dimension_numbers=(((1,), (0,)), ((), ())),
        preferred_element_type=jnp.float32)
    sum_ref[0] = summed


@jax.jit
def kernel(x, W, b):
    B, T, D = x.shape
    k_value = max(int(T * _TOPK_FRAC), 1)
    idx_bits = max((T - 1).bit_length(), 1)
    Wt = W.reshape(1, D)
    b2 = b.reshape(1, 1)

    body = functools.partial(_attn_body, k_value, idx_bits)
    summed, emph = pl.pallas_call(
        body,
        grid=(B,),
        in_specs=[
            pl.BlockSpec((1, D), lambda i: (0, 0)),
            pl.BlockSpec((1, 1), lambda i: (0, 0)),
            pl.BlockSpec((1, T, D), lambda i: (i, 0, 0)),
        ],
        out_specs=[
            pl.BlockSpec((1, 1, D), lambda i: (i, 0, 0)),
            pl.BlockSpec((1, 1, T), lambda i: (i, 0, 0)),
        ],
        out_shape=[
            jax.ShapeDtypeStruct((B, 1, D), jnp.float32),
            jax.ShapeDtypeStruct((B, 1, T), jnp.float32),
        ],
    )(Wt, b2, x)
    return summed.reshape(B, D), emph.reshape(B, T)


# speedup vs baseline: 1.9377x; 1.9377x over previous
import functools

import jax
import jax.numpy as jnp
from jax import lax
from jax.experimental import pallas as pl
from jax.experimental.pallas import tpu as pltpu

_EMPHASIS = 1.5
_TOPK_FRAC = 0.2


def _attn_body(nb, k_value, idx_bits, Wt_ref, b_ref, x_ref, sum_ref, emph_ref):
    int_min = jnp.int32(-(2 ** 31))
    Wt = Wt_ref[...]
    T = x_ref.shape[1]

    rows = [
        lax.dot_general(
            Wt, x_ref[b], dimension_numbers=(((1,), (1,)), ((), ())),
            preferred_element_type=jnp.float32)
        for b in range(nb)
    ]
    pre = jnp.concatenate(rows, axis=0)
    e = jnp.tanh(pre + b_ref[0, 0])
    ex = jnp.exp(e)
    z = jnp.sum(ex, axis=1, keepdims=True)

    bits = lax.bitcast_convert_type(e, jnp.int32)
    skey = jnp.where(bits >= 0, bits, bits ^ jnp.int32(0x7FFFFFFF))
    skey = jnp.where(bits == int_min, jnp.int32(0), skey)

    def val_round(i, p_u):
        bit = lax.shift_left(jnp.int32(1), jnp.int32(31) - i)
        cand = p_u | bit
        scand = cand ^ int_min
        c = jnp.sum((skey >= scand).astype(jnp.int32), axis=1, keepdims=True)
        return jnp.where(c >= k_value, cand, p_u)

    p_u = lax.fori_loop(0, 32, val_round, jnp.full((nb, 1), 0, jnp.int32))
    s_star = p_u ^ int_min

    gt = skey > s_star
    g = jnp.sum(gt.astype(jnp.int32), axis=1, keepdims=True)
    r = k_value - g
    eq = skey == s_star
    idx = lax.broadcasted_iota(jnp.int32, (nb, T), 1)

    def idx_round(j, p):
        bit = lax.shift_left(jnp.int32(1), jnp.int32(idx_bits - 1) - j)
        t_test = p | (bit - 1)
        f = jnp.sum((eq & (idx <= t_test)).astype(jnp.int32),
                    axis=1, keepdims=True)
        return jnp.where(f >= r, p, p | bit)

    t_star = lax.fori_loop(0, idx_bits, idx_round,
                           jnp.full((nb, 1), 0, jnp.int32))
    mask = gt | (eq & (idx <= t_star))

    w = jnp.where(mask, ex * jnp.float32(_EMPHASIS), ex) / z
    emph_ref[:, 0, :] = w
    for b in range(nb):
        sum_ref[b] = lax.dot_general(
            w[b:b + 1], x_ref[b], dimension_numbers=(((1,), (0,)), ((), ())),
            preferred_element_type=jnp.float32)


@jax.jit
def kernel(x, W, b):
    B, T, D = x.shape
    nb = 2
    k_value = max(int(T * _TOPK_FRAC), 1)
    idx_bits = max((T - 1).bit_length(), 1)
    Wt = W.reshape(1, D)
    b2 = b.reshape(1, 1)

    body = functools.partial(_attn_body, nb, k_value, idx_bits)
    summed, emph = pl.pallas_call(
        body,
        grid=(B // nb,),
        in_specs=[
            pl.BlockSpec((1, D), lambda i: (0, 0)),
            pl.BlockSpec((1, 1), lambda i: (0, 0)),
            pl.BlockSpec((nb, T, D), lambda i: (i, 0, 0)),
        ],
        out_specs=[
            pl.BlockSpec((nb, 1, D), lambda i: (i, 0, 0)),
            pl.BlockSpec((nb, 1, T), lambda i: (i, 0, 0)),
        ],
        out_shape=[
            jax.ShapeDtypeStruct((B, 1, D), jnp.float32),
            jax.ShapeDtypeStruct((B, 1, T), jnp.float32),
        ],
        compiler_params=pltpu.CompilerParams(
            vmem_limit_bytes=100 * 1024 * 1024,
        ),
    )(Wt, b2, x)
    return summed.reshape(B, D), emph.reshape(B, T)


# speedup vs baseline: 1.9417x; 1.0021x over previous
import functools

import jax
import jax.numpy as jnp
from jax import lax
from jax.experimental import pallas as pl
from jax.experimental.pallas import tpu as pltpu

_EMPHASIS = 1.5
_TOPK_FRAC = 0.2


def _attn_body(nb, nq, k_value, idx_bits, Wt_ref, b_ref, *refs):
    x_refs = refs[:nq]
    sum_ref, emph_ref = refs[nq], refs[nq + 1]
    int_min = jnp.int32(-(2 ** 31))
    Wt = Wt_ref[...]
    tq = x_refs[0].shape[1]
    T = tq * nq

    rows = [
        jnp.concatenate([
            lax.dot_general(
                Wt, x_refs[q][b], dimension_numbers=(((1,), (1,)), ((), ())),
                preferred_element_type=jnp.float32)
            for q in range(nq)
        ], axis=1)
        for b in range(nb)
    ]
    pre = jnp.concatenate(rows, axis=0)
    e = jnp.tanh(pre + b_ref[0, 0])
    ex = jnp.exp(e)
    z = jnp.sum(ex, axis=1, keepdims=True)

    bits = lax.bitcast_convert_type(e, jnp.int32)
    skey = jnp.where(bits >= 0, bits, bits ^ jnp.int32(0x7FFFFFFF))
    skey = jnp.where(bits == int_min, jnp.int32(0), skey)

    def val_round(i, p_u):
        bit = lax.shift_left(jnp.int32(1), jnp.int32(31) - i)
        cand = p_u | bit
        scand = cand ^ int_min
        c = jnp.sum((skey >= scand).astype(jnp.int32), axis=1, keepdims=True)
        return jnp.where(c >= k_value, cand, p_u)

    p_u = lax.fori_loop(0, 32, val_round, jnp.full((nb, 1), 0, jnp.int32))
    s_star = p_u ^ int_min

    gt = skey > s_star
    g = jnp.sum(gt.astype(jnp.int32), axis=1, keepdims=True)
    r = k_value - g
    eq = skey == s_star
    idx = lax.broadcasted_iota(jnp.int32, (nb, T), 1)

    def idx_round(j, p):
        bit = lax.shift_left(jnp.int32(1), jnp.int32(idx_bits - 1) - j)
        t_test = p | (bit - 1)
        f = jnp.sum((eq & (idx <= t_test)).astype(jnp.int32),
                    axis=1, keepdims=True)
        return jnp.where(f >= r, p, p | bit)

    t_star = lax.fori_loop(0, idx_bits, idx_round,
                           jnp.full((nb, 1), 0, jnp.int32))
    mask = gt | (eq & (idx <= t_star))

    w = jnp.where(mask, ex * jnp.float32(_EMPHASIS), ex) / z
    emph_ref[:, 0, :] = w
    for b in range(nb):
        parts = [
            lax.dot_general(
                w[b:b + 1, q * tq:(q + 1) * tq], x_refs[q][b],
                dimension_numbers=(((1,), (0,)), ((), ())),
                preferred_element_type=jnp.float32)
            for q in range(nq)
        ]
        acc = parts[0]
        for p in parts[1:]:
            acc = acc + p
        sum_ref[b] = acc


@jax.jit
def kernel(x, W, b):
    B, T, D = x.shape
    nb = 2
    nq = 4
    tq = T // nq
    k_value = max(int(T * _TOPK_FRAC), 1)
    idx_bits = max((T - 1).bit_length(), 1)
    Wt = W.reshape(1, D)
    b2 = b.reshape(1, 1)

    def make_xspec(q):
        return pl.BlockSpec((nb, tq, D), lambda i, q=q: (i, q, 0))

    body = functools.partial(_attn_body, nb, nq, k_value, idx_bits)
    summed, emph = pl.pallas_call(
        body,
        grid=(B // nb,),
        in_specs=[
            pl.BlockSpec((1, D), lambda i: (0, 0)),
            pl.BlockSpec((1, 1), lambda i: (0, 0)),
        ] + [make_xspec(q) for q in range(nq)],
        out_specs=[
            pl.BlockSpec((nb, 1, D), lambda i: (i, 0, 0)),
            pl.BlockSpec((nb, 1, T), lambda i: (i, 0, 0)),
        ],
        out_shape=[
            jax.ShapeDtypeStruct((B, 1, D), jnp.float32),
            jax.ShapeDtypeStruct((B, 1, T), jnp.float32),
        ],
        compiler_params=pltpu.CompilerParams(
            vmem_limit_bytes=100 * 1024 * 1024,
        ),
    )(Wt, b2, *([x] * nq))
    return summed.reshape(B, D), emph.reshape(B, T)


# speedup vs baseline: 2.7787x; 1.4310x over previous
import functools

import jax
import jax.numpy as jnp
from jax import lax
from jax.experimental import pallas as pl
from jax.experimental.pallas import tpu as pltpu

_EMPHASIS = 1.5
_TOPK_FRAC = 0.2


def _attn_body(nb, k_value, Wt_ref, b_ref, x_ref, sum_ref, emph_ref, xbf_ref):
    int_min = jnp.int32(-(2 ** 31))
    T = x_ref.shape[1]

    xbf_ref[...] = x_ref[...].astype(jnp.bfloat16)
    Wt = Wt_ref[...]

    rows = [
        lax.dot_general(
            Wt, x_ref[b], dimension_numbers=(((1,), (1,)), ((), ())),
            preferred_element_type=jnp.float32)
        for b in range(nb)
    ]
    pre = jnp.concatenate(rows, axis=0)[:, None, :]
    e = jnp.tanh(pre + b_ref[0, 0])
    ex = jnp.exp(e)
    z = jnp.sum(ex, axis=2, keepdims=True)

    bits = lax.bitcast_convert_type(e, jnp.int32)
    skey = jnp.where(bits >= 0, bits, bits ^ jnp.int32(0x7FFFFFFF))
    skey = jnp.where(bits == int_min, jnp.int32(0), skey)

    jv15 = lax.broadcasted_iota(jnp.int32, (nb, 15, 1), 1) + 1
    p_u = jnp.zeros((nb, 1, 1), jnp.int32)
    for rnd in range(8):
        shift = 28 - 4 * rnd
        cand = p_u | lax.shift_left(jv15, shift)
        scand = cand ^ int_min
        cmp = (skey >= scand).astype(jnp.int32)
        c = jnp.sum(cmp, axis=2, keepdims=True)
        j_star = jnp.sum((c >= k_value).astype(jnp.int32),
                         axis=1, keepdims=True)
        p_u = p_u | lax.shift_left(j_star, shift)
    s_star = p_u ^ int_min

    gt = skey > s_star
    g = jnp.sum(gt.astype(jnp.int32), axis=2, keepdims=True)
    r = k_value - g
    eq = skey == s_star
    idx = lax.broadcasted_iota(jnp.int32, (nb, 1, T), 2)
    jv16 = lax.broadcasted_iota(jnp.int32, (nb, 16, 1), 1)
    p_i = jnp.zeros((nb, 1, 1), jnp.int32)
    for sh in (8, 4, 0):
        low = (1 << sh) - 1
        t_test = p_i | lax.shift_left(jv16, sh) | low
        hit = (eq & (idx <= t_test)).astype(jnp.int32)
        f = jnp.sum(hit, axis=2, keepdims=True)
        n_star = jnp.sum((f < r).astype(jnp.int32),
                         axis=1, keepdims=True)
        p_i = p_i | lax.shift_left(n_star, sh)
    mask = gt | (eq & (idx <= p_i))

    w = jnp.where(mask, ex * jnp.float32(_EMPHASIS), ex) / z
    emph_ref[...] = w
    wbf = w.astype(jnp.bfloat16)
    for b in range(nb):
        sum_ref[b] = lax.dot_general(
            wbf[b], xbf_ref[b], dimension_numbers=(((1,), (0,)), ((), ())),
            preferred_element_type=jnp.float32)


@jax.jit
def kernel(x, W, b):
    B, T, D = x.shape
    nb = 2
    k_value = max(int(T * _TOPK_FRAC), 1)
    Wt = W.reshape(1, D)
    b2 = b.reshape(1, 1)

    body = functools.partial(_attn_body, nb, k_value)
    summed, emph = pl.pallas_call(
        body,
        grid=(B // nb,),
        in_specs=[
            pl.BlockSpec((1, D), lambda i: (0, 0)),
            pl.BlockSpec((1, 1), lambda i: (0, 0)),
            pl.BlockSpec((nb, T, D), lambda i: (i, 0, 0)),
        ],
        out_specs=[
            pl.BlockSpec((nb, 1, D), lambda i: (i, 0, 0)),
            pl.BlockSpec((nb, 1, T), lambda i: (i, 0, 0)),
        ],
        out_shape=[
            jax.ShapeDtypeStruct((B, 1, D), jnp.float32),
            jax.ShapeDtypeStruct((B, 1, T), jnp.float32),
        ],
        scratch_shapes=[pltpu.VMEM((nb, T, D), jnp.bfloat16)],
        compiler_params=pltpu.CompilerParams(
            vmem_limit_bytes=100 * 1024 * 1024,
        ),
    )(Wt, b2, x)
    return summed.reshape(B, D), emph.reshape(B, T)
